# baseline (speedup 1.0000x reference)
import jax
import jax.numpy as jnp
from jax import lax
from jax.experimental import pallas as pl
from jax.experimental.pallas import tpu as pltpu
from jax.experimental.layout import Format, Layout

TOKENS = 32768
DIM = 768
NUM_EXPERTS = 64
TOP_K = 2
BLOCK = 4096


def _gate_block(x_ref, w_ref, b_ref, out_ref, idx_ref):
    xb = x_ref[...]
    logits = lax.dot_general(
        xb, w_ref[...], (((1,), (1,)), ((), ())),
        preferred_element_type=jnp.float32,
    )
    logits = logits + b_ref[...]

    iota = lax.broadcasted_iota(jnp.int32, logits.shape, 1).astype(jnp.float32)
    neg_inf = jnp.float32(-jnp.inf)
    big = jnp.float32(NUM_EXPERTS)

    v1 = jnp.max(logits, axis=1, keepdims=True)
    i1 = jnp.min(jnp.where(logits == v1, iota, big), axis=1, keepdims=True)
    hit1 = iota == i1
    masked = jnp.where(hit1, neg_inf, logits)
    v2 = jnp.max(masked, axis=1, keepdims=True)
    i2 = jnp.min(jnp.where(masked == v2, iota, big), axis=1, keepdims=True)
    hit2 = iota == i2

    e2 = jnp.exp(v2 - v1)
    denom = 1.0 + e2
    p1 = 1.0 / denom
    p2 = e2 / denom

    out_ref[...] = jnp.where(hit1, p1, jnp.where(hit2, p2, 0.0))
    pair = jnp.concatenate([i1, i2], axis=1).astype(jnp.int32)
    idx_ref[...] = pair.T


def _gate(x, W, b):
    b2 = b.reshape(1, NUM_EXPERTS)
    grid = (TOKENS // BLOCK,)
    out, idx_t = pl.pallas_call(
        _gate_block,
        grid=grid,
        in_specs=[
            pl.BlockSpec((BLOCK, DIM), lambda i: (i, 0)),
            pl.BlockSpec((NUM_EXPERTS, DIM), lambda i: (0, 0)),
            pl.BlockSpec((1, NUM_EXPERTS), lambda i: (0, 0)),
        ],
        out_specs=[
            pl.BlockSpec((BLOCK, NUM_EXPERTS), lambda i: (i, 0)),
            pl.BlockSpec((TOP_K, BLOCK), lambda i: (0, i)),
        ],
        out_shape=[
            jax.ShapeDtypeStruct((TOKENS, NUM_EXPERTS), jnp.float32),
            jax.ShapeDtypeStruct((TOP_K, TOKENS), jnp.int32),
        ],
        compiler_params=pltpu.CompilerParams(
            dimension_semantics=("parallel",),
        ),
    )(x, W, b2)
    return (out, idx_t.T)


kernel = jax.jit(_gate)
